# Initial kernel scaffold; baseline (speedup 1.0000x reference)
import jax
import jax.numpy as jnp
from jax.experimental import pallas as pl


def kernel(node_feats, g_feats, segment_ids, W1, b1, W2, b2, Wih, Whh, bih, bhh):
    raise NotImplementedError("write your pallas kernel here")


# speedup vs baseline: 5.3384x; 5.3384x over previous
import functools

import jax
import jax.numpy as jnp
from jax import lax
from jax.experimental import pallas as pl
from jax.experimental.pallas import tpu as pltpu
from jax.experimental.pallas import tpu_sc as plsc

N = 50000
B = 1024
F = 256
TILE = 80
NT = N // TILE
NW = 32
ROUNDS = (NT + NW - 1) // NW
GROUPS = TILE // 16
KV = F // 16
ROW_BLK = 1000


def _matvec_body(node_ref, w_ref, v_ref, vmax_ref):
    x = jnp.dot(node_ref[...], w_ref[...], preferred_element_type=jnp.float32)
    v_ref[...] = x
    m = jnp.max(x, keepdims=True)
    pid = pl.program_id(0)

    @pl.when(pid == 0)
    def _():
        vmax_ref[...] = m

    @pl.when(pid != 0)
    def _():
        vmax_ref[...] = jnp.maximum(vmax_ref[...], m)


def _matvec(node_feats, w1b):
    return pl.pallas_call(
        _matvec_body,
        grid=(N // ROW_BLK,),
        in_specs=[
            pl.BlockSpec((ROW_BLK, F), lambda i: (i, 0)),
            pl.BlockSpec((F, 1), lambda i: (0, 0)),
        ],
        out_specs=[
            pl.BlockSpec((ROW_BLK, 1), lambda i: (i, 0)),
            pl.BlockSpec((1, 1), lambda i: (0, 0)),
        ],
        out_shape=[
            jax.ShapeDtypeStruct((N, 1), jnp.float32),
            jax.ShapeDtypeStruct((1, 1), jnp.float32),
        ],
    )(node_feats, w1b)


def _umat_body(g_ref, w_ref, u_ref, umax_ref):
    x = jnp.dot(jnp.maximum(g_ref[...], 0.0), w_ref[...],
                preferred_element_type=jnp.float32)
    u_ref[...] = x
    umax_ref[...] = jnp.max(x, keepdims=True)


def _umat(g_feats, w1a):
    return pl.pallas_call(
        _umat_body,
        out_shape=[
            jax.ShapeDtypeStruct((B, 1), jnp.float32),
            jax.ShapeDtypeStruct((1, 1), jnp.float32),
        ],
    )(g_feats, w1a)


def _sc_body(node_hbm, v_hbm, seg_hbm, u_hbm, c_hbm, zp_hbm, zd_hbm,
             pooled_out, d_out,
             rows, vb, segb, eb, erows, ub, cb, shp, shd):
    cid = lax.axis_index("c")
    sid = lax.axis_index("s")
    wid = sid * 2 + cid

    pltpu.sync_copy(u_hbm, ub)
    pltpu.sync_copy(c_hbm, cb)
    zero16 = jnp.zeros((16,), jnp.float32)
    for i in range(TILE):
        erows[i, :] = zero16

    @pl.when(sid == 0)
    def _():
        pltpu.sync_copy(zp_hbm, shp)
        pltpu.sync_copy(zd_hbm, shd)

    plsc.subcore_barrier()

    cv = cb[...]
    b1s = cv[0]
    shift = cv[1]
    lanes = lax.iota(jnp.int32, 16)

    def round_body(r, carry):
        t = r * NW + wid

        @pl.when(t < NT)
        def _():
            base = t * TILE
            pltpu.sync_copy(node_hbm.at[pl.ds(base, TILE)], rows)
            pltpu.sync_copy(v_hbm.at[pl.ds(base, TILE)], vb)
            pltpu.sync_copy(seg_hbm.at[pl.ds(base, TILE)], segb)
            for g in range(GROUPS):
                sv = segb[pl.ds(g * 16, 16)]
                vv = vb[pl.ds(g * 16, 16)]
                uu = plsc.load_gather(ub, [sv])
                zz = vv + uu + b1s
                zz = jnp.where(zz > 0.0, zz, zz * 0.01)
                ee = jnp.exp(zz - shift)
                eb[pl.ds(g * 16, 16)] = ee
                plsc.store_scatter(erows, [g * 16 + lanes,
                                           jnp.zeros((16,), jnp.int32)], ee)

            def node_body(n, c2):
                es = plsc.load_gather(eb, [jnp.full((16,), n, jnp.int32)])
                for k in range(KV):
                    rows[n, pl.ds(k * 16, 16)] = rows[n, pl.ds(k * 16, 16)] * es
                return c2

            lax.fori_loop(0, TILE, node_body, 0)
            pltpu.sync_copy(rows, shp.at[segb], add=True)
            pltpu.sync_copy(erows, shd.at[segb], add=True)

        return carry

    lax.fori_loop(0, ROUNDS, round_body, 0)
    plsc.subcore_barrier()

    @pl.when(sid == 0)
    def _():
        pltpu.sync_copy(shp, pooled_out.at[cid])
        pltpu.sync_copy(shd, d_out.at[cid])


def _sc_pool(node_feats, v, seg, u, consts, zp, zd):
    mesh = plsc.VectorSubcoreMesh(core_axis_name="c", subcore_axis_name="s")
    kern = pl.kernel(
        _sc_body,
        mesh=mesh,
        compiler_params=pltpu.CompilerParams(
            needs_layout_passes=False, use_tc_tiling_on_sc=False),
        out_type=[
            jax.ShapeDtypeStruct((2, B, F), jnp.float32),
            jax.ShapeDtypeStruct((2, B, 16), jnp.float32),
        ],
        scratch_types=[
            pltpu.VMEM((TILE, F), jnp.float32),
            pltpu.VMEM((TILE,), jnp.float32),
            pltpu.VMEM((TILE,), jnp.int32),
            pltpu.VMEM((TILE,), jnp.float32),
            pltpu.VMEM((TILE, 16), jnp.float32),
            pltpu.VMEM((B,), jnp.float32),
            pltpu.VMEM((16,), jnp.float32),
            pltpu.VMEM_SHARED((B, F), jnp.float32),
            pltpu.VMEM_SHARED((B, 16), jnp.float32),
        ],
    )
    return kern(node_feats, v, seg, u, consts, zp, zd)


def _final_body(pp_ref, dp_ref, g_ref, W2_ref, b2_ref, Wih_ref, Whh_ref,
                bih_ref, bhh_ref, out_ref):
    pooled = pp_ref[0] + pp_ref[1]
    d = dp_ref[0, :, 0] + dp_ref[1, :, 0]
    nonempty = d > 0.0
    inv = jnp.where(nonempty, 1.0 / jnp.where(nonempty, d, 1.0), 0.0)
    ctx_in = pooled * inv[:, None]
    dn = (((1,), (1,)), ((), ()))
    g_repr = lax.dot_general(ctx_in, W2_ref[...], dn,
                             preferred_element_type=jnp.float32)
    g_repr = g_repr + b2_ref[...][None, :] * nonempty[:, None].astype(jnp.float32)
    context = jnp.where(g_repr > 0.0, g_repr,
                        jnp.exp(jnp.minimum(g_repr, 0.0)) - 1.0)
    g = g_ref[...]
    gi = lax.dot_general(context, Wih_ref[...], dn,
                         preferred_element_type=jnp.float32) + bih_ref[...][None, :]
    gh = lax.dot_general(g, Whh_ref[...], dn,
                         preferred_element_type=jnp.float32) + bhh_ref[...][None, :]
    i_r, i_z, i_n = gi[:, :F], gi[:, F:2 * F], gi[:, 2 * F:]
    h_r, h_z, h_n = gh[:, :F], gh[:, F:2 * F], gh[:, 2 * F:]
    r = jax.nn.sigmoid(i_r + h_r)
    uu = jax.nn.sigmoid(i_z + h_z)
    n = jnp.tanh(i_n + r * h_n)
    out_ref[...] = (1.0 - uu) * n + uu * g


def _final(pooled_parts, d_parts, g_feats, W2, b2, Wih, Whh, bih, bhh):
    return pl.pallas_call(
        _final_body,
        out_shape=jax.ShapeDtypeStruct((B, F), jnp.float32),
    )(pooled_parts, d_parts, g_feats, W2, b2, Wih, Whh, bih, bhh)


def kernel(node_feats, g_feats, segment_ids, W1, b1, W2, b2, Wih, Whh, bih, bhh):
    w1a = W1[0, :F].reshape(F, 1)
    w1b = W1[0, F:].reshape(F, 1)
    v2d, vmax = _matvec(node_feats, w1b)
    u2d, umax = _umat(g_feats, w1a)
    M = jnp.maximum(vmax[0, 0] + umax[0, 0] + b1[0], 0.0)
    consts = jnp.concatenate(
        [b1, M[None], jnp.zeros((14,), jnp.float32)]).astype(jnp.float32)
    zp = jnp.zeros((B, F), jnp.float32)
    zd = jnp.zeros((B, 16), jnp.float32)
    pooled_parts, d_parts = _sc_pool(
        node_feats, v2d.reshape(N), segment_ids, u2d.reshape(B), consts, zp, zd)
    return _final(pooled_parts, d_parts, g_feats, W2, b2, Wih, Whh, bih, bhh)
